# baseline (speedup 1.0000x reference)
import functools

import jax
import jax.numpy as jnp
from jax import lax
from jax.experimental import pallas as pl
from jax.experimental.pallas import tpu as pltpu
from jax.experimental.pallas import tpu_sc as plsc

DIM = 1024
B, S = 4, 2048
TOK = B * S
EPS = 1e-5
LANES = 16
J = DIM // LANES

NC, NS = 2, 16
NW = NC * NS
TPW = TOK // NW
C = 32
NCHUNK = TPW // C


def _rsqrt_newton(x):
    i = lax.bitcast_convert_type(x, jnp.int32)
    i = jnp.int32(0x5F3759DF) - lax.shift_right_arithmetic(i, 1)
    y = lax.bitcast_convert_type(i, jnp.float32)
    for _ in range(3):
        y = y * (1.5 - 0.5 * x * y * y)
    return y


def _emb_ln_body(ids_hbm, pos_hbm, gamma_hbm, beta_hbm, table_hbm, out_hbm,
                 ids_v, wbuf, pbuf, g_v, b_v, r_s, m_s, sem):
    wid = lax.axis_index("s") * NC + lax.axis_index("c")
    base = wid * TPW
    s0 = lax.rem(base, S)

    pltpu.sync_copy(ids_hbm.at[pl.ds(base, TPW)], ids_v)
    pltpu.sync_copy(gamma_hbm, g_v)
    pltpu.sync_copy(beta_hbm, b_v)

    def chunk_body(k, carry):
        t0 = k * C
        pltpu.async_copy(table_hbm.at[ids_v.at[pl.ds(t0, C)]], wbuf, sem).wait()
        pltpu.sync_copy(pos_hbm.at[pl.ds(s0 + t0, C)], pbuf)

        def tok_stats(t, c):
            acc = jnp.zeros((LANES,), jnp.float32)
            acc2 = jnp.zeros((LANES,), jnp.float32)
            for j in range(J):
                sl = pl.ds(j * LANES, LANES)
                x = wbuf[t, sl] + pbuf[t, sl]
                wbuf[t, sl] = x
                acc = acc + x
                acc2 = acc2 + x * x
            tot = jnp.sum(acc)
            tot2 = jnp.sum(acc2)
            mu = tot * (1.0 / DIM)
            var = tot2 * (1.0 / DIM) - mu * mu
            r = _rsqrt_newton(var + EPS)
            r_s[t] = r
            m_s[t] = mu * r
            return c

        lax.fori_loop(0, C, tok_stats, 0)

        def col_apply(j, c):
            sl = pl.ds(j * LANES, LANES)
            g = g_v[sl]
            b = b_v[sl]

            def tok_apply(t, cc):
                rt = r_s[t]
                mt = m_s[t]
                x = wbuf[t, sl]
                wbuf[t, sl] = x * (g * rt) + (b - mt * g)
                return cc

            lax.fori_loop(0, C, tok_apply, 0, unroll=4)
            return c

        lax.fori_loop(0, J, col_apply, 0)

        pltpu.sync_copy(wbuf, out_hbm.at[pl.ds(base + t0, C)])
        return carry

    lax.fori_loop(0, NCHUNK, chunk_body, 0)


@jax.jit
def kernel(input_ids, x_qkv, word_table, pos_table, gamma, beta):
    del x_qkv
    ids_flat = input_ids.reshape(TOK).astype(jnp.int32)

    mesh = plsc.VectorSubcoreMesh(
        core_axis_name="c", subcore_axis_name="s",
        num_cores=NC, num_subcores=NS)

    run = pl.kernel(
        _emb_ln_body,
        out_type=jax.ShapeDtypeStruct((TOK, DIM), jnp.float32),
        mesh=mesh,
        compiler_params=pltpu.CompilerParams(needs_layout_passes=False),
        scratch_types=[
            pltpu.VMEM((TPW,), jnp.int32),
            pltpu.VMEM((C, DIM), jnp.float32),
            pltpu.VMEM((C, DIM), jnp.float32),
            pltpu.VMEM((DIM,), jnp.float32),
            pltpu.VMEM((DIM,), jnp.float32),
            pltpu.SMEM((C,), jnp.float32),
            pltpu.SMEM((C,), jnp.float32),
            pltpu.SemaphoreType.DMA,
        ],
    )
    out = run(ids_flat, pos_table, gamma, beta, word_table)
    return out.reshape(B, S, DIM)
